# baseline (speedup 1.0000x reference)
import functools

import jax
import jax.numpy as jnp
from jax import lax
from jax.experimental import pallas as pl
from jax.experimental.pallas import tpu as pltpu
from jax.experimental.pallas import tpu_sc as plsc

N_NODES = 10000
N_EDGES = 320000

NC = 2
NS = 16
NW = NC * NS
CHUNK = 128
N_PAD = 10240
EPW = (N_EDGES // NW // CHUNK + 1) * CHUNK
E_PAD = EPW * NW
NCH = EPW // CHUNK
ROWS_PER_TILE = N_PAD // NS
BUFS = 8
RB = 640
GRID = N_PAD // RB

_mesh = plsc.VectorSubcoreMesh(core_axis_name="c", subcore_axis_name="s")


def _zero16():
    return jnp.zeros((16,), jnp.float32)


@functools.partial(
    pl.kernel,
    out_type=jax.ShapeDtypeStruct((NC, N_PAD), jnp.float32),
    mesh=_mesh,
    compiler_params=pltpu.CompilerParams(use_tc_tiling_on_sc=False),
    scratch_types=[
        pltpu.VMEM((NCH, CHUNK), jnp.int32),
        pltpu.VMEM((CHUNK,), jnp.float32),
        pltpu.VMEM((ROWS_PER_TILE,), jnp.float32),
        pltpu.VMEM_SHARED((N_PAD,), jnp.float32),
    ],
)
def _deg_kernel(e_hbm, out_hbm, dst_v, ones_v, stage_v, acc_sh):
    c = lax.axis_index("c")
    s = lax.axis_index("s")
    wid = s * NC + c

    def fill(i, _):
        ones_v[pl.ds(i * 16, 16)] = _zero16() + 1.0
        return 0
    lax.fori_loop(0, CHUNK // 16, fill, 0)

    def zst(i, _):
        stage_v[pl.ds(i * 16, 16)] = _zero16()
        return 0
    lax.fori_loop(0, ROWS_PER_TILE // 16, zst, 0)
    pltpu.sync_copy(stage_v, acc_sh.at[pl.ds(s * ROWS_PER_TILE, ROWS_PER_TILE)])
    plsc.subcore_barrier()

    pltpu.sync_copy(e_hbm.at[1, pl.ds(wid * NCH, NCH)], dst_v)

    def body(ch, _):
        pltpu.sync_copy(ones_v, acc_sh.at[dst_v.at[ch]], add=True)
        return 0
    lax.fori_loop(0, NCH, body, 0)
    plsc.subcore_barrier()

    pltpu.sync_copy(acc_sh.at[pl.ds(s * ROWS_PER_TILE, ROWS_PER_TILE)], stage_v)
    pltpu.sync_copy(stage_v, out_hbm.at[c, pl.ds(s * ROWS_PER_TILE, ROWS_PER_TILE)])


def _make_agg(F):
    ncp = 1

    @functools.partial(
        pl.kernel,
        out_type=jax.ShapeDtypeStruct((NC, N_PAD, F), jnp.float32),
        mesh=_mesh,
        compiler_params=pltpu.CompilerParams(use_tc_tiling_on_sc=False),
        scratch_types=[
            pltpu.VMEM((NCH, CHUNK), jnp.int32),
            pltpu.VMEM((NCH, CHUNK), jnp.int32),
            [pltpu.VMEM((CHUNK, F), jnp.float32) for _ in range(BUFS)],
            pltpu.VMEM((ROWS_PER_TILE, F), jnp.float32),
            pltpu.VMEM((ROWS_PER_TILE, F), jnp.float32),
            pltpu.VMEM_SHARED((ncp, N_PAD, F), jnp.float32),
            [pltpu.SemaphoreType.DMA for _ in range(BUFS)],
            [pltpu.SemaphoreType.DMA for _ in range(BUFS)],
        ],
    )
    def agg(e_hbm, xp_hbm, out_hbm,
            src_v, dst_v, rows, stage_v, tmp_v, acc_sh, gsems, ssems):
        c = lax.axis_index("c")
        s = lax.axis_index("s")
        wid = s * NC + c
        grp = lax.rem(s, ncp)

        def zr(i, _):
            r = i // (F // 16)
            j = i % (F // 16)
            rows[0][r, pl.ds(j * 16, 16)] = _zero16()
            return 0
        lax.fori_loop(0, CHUNK * F // 16, zr, 0)
        for kc in range(ncp):
            for k in range(ROWS_PER_TILE // CHUNK):
                pltpu.sync_copy(
                    rows[0],
                    acc_sh.at[kc, pl.ds(s * ROWS_PER_TILE + k * CHUNK, CHUNK)])
        plsc.subcore_barrier()

        pltpu.sync_copy(e_hbm.at[0, pl.ds(wid * NCH, NCH)], src_v)
        pltpu.sync_copy(e_hbm.at[1, pl.ds(wid * NCH, NCH)], dst_v)

        def gather(ch, b):
            return pltpu.async_copy(xp_hbm.at[src_v.at[ch]], rows[b], gsems[b])

        def scat(ch, b):
            dst = acc_sh.at[grp].at[dst_v.at[ch]]
            return pltpu.async_copy(rows[b], dst, ssems[b], add=True)

        LAG = 4
        descs = {}
        sdescs = {}
        for b in range(LAG):
            descs[b] = gather(b, b)
        for ch in range(NCH):
            b = ch % BUFS
            la = ch + LAG
            if la < NCH:
                bl = la % BUFS
                if bl in sdescs:
                    sdescs.pop(bl).wait()
                descs[bl] = gather(la, bl)
            descs[b].wait()
            sdescs[b] = scat(ch, b)
        for d in sdescs.values():
            d.wait()
        plsc.subcore_barrier()

        rowslice = pl.ds(s * ROWS_PER_TILE, ROWS_PER_TILE)
        pltpu.sync_copy(acc_sh.at[0, rowslice], stage_v)
        for kc in range(1, ncp):
            pltpu.sync_copy(acc_sh.at[kc, rowslice], tmp_v)

            def addk(i, _):
                r = i // (F // 16)
                j = i % (F // 16)
                sl = pl.ds(j * 16, 16)
                stage_v[r, sl] = stage_v[r, sl] + tmp_v[r, sl]
                return 0
            lax.fori_loop(0, ROWS_PER_TILE * F // 16, addk, 0)
        pltpu.sync_copy(stage_v, out_hbm.at[c, rowslice])

    return agg


_agg32 = _make_agg(32)
_agg16 = _make_agg(16)


def _iota2(shape, dim):
    return lax.broadcasted_iota(jnp.int32, shape, dim)


def _dinv_col(deg_ref):
    deg = deg_ref[0:1, :] + deg_ref[1:2, :] + 1.0
    return jnp.transpose(lax.rsqrt(deg), (1, 0))


def _pack_rows(y, p, f):
    rr = RB // p
    r = _iota2((rr, RB), 0)
    n = _iota2((rr, RB), 1)
    outs = []
    for g in range(p):
        sel = jnp.where(n == p * r + g, 1.0, 0.0)
        outs.append(jnp.dot(sel, y, preferred_element_type=jnp.float32))
    return jnp.concatenate(outs, axis=1)


def _unpack_rows(pk, p, f):
    rr = RB // p
    n = _iota2((RB, rr), 0)
    r = _iota2((RB, rr), 1)
    acc = None
    for g in range(p):
        sel = jnp.where((r == n // p) & (n % p == g), 1.0, 0.0)
        t = jnp.dot(sel, pk[:, g * f:(g + 1) * f],
                    preferred_element_type=jnp.float32)
        acc = t if acc is None else acc + t
    return acc


def _prep_body(x_ref, w_ref, deg_ref, out_ref):
    dcol = _dinv_col(deg_ref)
    y = jnp.dot(x_ref[...], w_ref[...], preferred_element_type=jnp.float32)
    out_ref[...] = _pack_rows(y * dcol, 4, 32)


def _make_mid_body(p_in, f_in, p_out, f_out):
    def body(p_ref, xp_ref, deg_ref, b_ref, w_ref, out_ref):
        dcol = _dinv_col(deg_ref)
        tot = _unpack_rows(p_ref[0] + p_ref[1] + xp_ref[...], p_in, f_in)
        h = jnp.maximum(dcol * tot + b_ref[...], 0.0)
        y = jnp.dot(h, w_ref[...], preferred_element_type=jnp.float32) * dcol
        out_ref[...] = _pack_rows(y, p_out, f_out)
    return body


def _final_body(p_ref, xp_ref, deg_ref, b_ref, out_ref):
    dcol = _dinv_col(deg_ref)
    tot = _unpack_rows(p_ref[0] + p_ref[1] + xp_ref[...], 8, 16)
    z = dcol * tot + b_ref[...]
    col = _iota2(z.shape, 1)
    valid = col < 11
    zm = jnp.where(valid, z, -jnp.inf)
    m = jnp.max(zm, axis=1, keepdims=True)
    e = jnp.where(valid, jnp.exp(z - m), 0.0)
    out_ref[...] = z - m - jnp.log(jnp.sum(e, axis=1, keepdims=True))


def _deg_spec():
    return pl.BlockSpec((2, RB), lambda i: (0, i))


def _pk_spec(p):
    return pl.BlockSpec((RB // p * 128 // 128, 128), lambda i: (i, 0))


def _parts_pk_spec(p):
    return pl.BlockSpec((NC, RB // p, 128), lambda i: (0, i, 0))


def _full_spec(shape):
    return pl.BlockSpec(shape, lambda i: tuple(0 for _ in shape))


def _tc_prep(x_pad, W1, deg):
    return pl.pallas_call(
        _prep_body,
        grid=(GRID,),
        in_specs=[pl.BlockSpec((RB, 128), lambda i: (i, 0)),
                  _full_spec((128, 32)), _deg_spec()],
        out_specs=_pk_spec(4),
        out_shape=jax.ShapeDtypeStruct((N_PAD // 4, 128), jnp.float32),
    )(x_pad, W1, deg)


def _tc_mid(parts_pk, xp_pk, deg, b, Wn, p_in, f_in, p_out, f_out):
    return pl.pallas_call(
        _make_mid_body(p_in, f_in, p_out, f_out),
        grid=(GRID,),
        in_specs=[_parts_pk_spec(p_in), _pk_spec(p_in), _deg_spec(),
                  _full_spec((1, f_in)), _full_spec(Wn.shape)],
        out_specs=_pk_spec(p_out),
        out_shape=jax.ShapeDtypeStruct((N_PAD // p_out, 128), jnp.float32),
    )(parts_pk, xp_pk, deg, b.reshape(1, f_in), Wn)


def _tc_final(parts_pk, xp_pk, deg, b):
    return pl.pallas_call(
        _final_body,
        grid=(GRID,),
        in_specs=[_parts_pk_spec(8), _pk_spec(8), _deg_spec(),
                  _full_spec((1, 16))],
        out_specs=pl.BlockSpec((RB, 16), lambda i: (i, 0)),
        out_shape=jax.ShapeDtypeStruct((N_PAD, 16), jnp.float32),
    )(parts_pk, xp_pk, deg, b.reshape(1, 16))


@jax.jit
def kernel(x, edge_index, W1, b1, W2, b2, W3, b3):
    ei = edge_index.astype(jnp.int32)
    e3 = jnp.pad(ei, ((0, 0), (0, E_PAD - N_EDGES)),
                 constant_values=N_NODES).reshape(2, NW * NCH, CHUNK)

    x_pad = jnp.pad(x, ((0, N_PAD - N_NODES), (0, 0)))
    W3p = jnp.pad(W3, ((0, 0), (0, 5)))
    b3p = jnp.pad(b3, (0, 5))

    deg = _deg_kernel(e3)

    x1p = _tc_prep(x_pad, W1, deg)
    parts1 = _agg32(e3, x1p.reshape(N_PAD, 32))
    x2p = _tc_mid(parts1.reshape(2, N_PAD // 4, 128), x1p, deg,
                  b1, W2, 4, 32, 8, 16)
    parts2 = _agg16(e3, x2p.reshape(N_PAD, 16))
    x3p = _tc_mid(parts2.reshape(2, N_PAD // 8, 128), x2p, deg,
                  b2, W3p, 8, 16, 8, 16)
    parts3 = _agg16(e3, x3p.reshape(N_PAD, 16))
    out = _tc_final(parts3.reshape(2, N_PAD // 8, 128), x3p, deg, b3p)
    return out[:N_NODES, :11]


# speedup vs baseline: 1.0110x; 1.0110x over previous
import functools

import jax
import jax.numpy as jnp
from jax import lax
from jax.experimental import pallas as pl
from jax.experimental.pallas import tpu as pltpu
from jax.experimental.pallas import tpu_sc as plsc

N_NODES = 10000
N_EDGES = 320000

NC = 2
NS = 16
NW = NC * NS
CHUNK = 128
N_PAD = 10240
EPW = (N_EDGES // NW // CHUNK + 1) * CHUNK
E_PAD = EPW * NW
NCH = EPW // CHUNK
ROWS_PER_TILE = N_PAD // NS
BUFS = 8
RB = 640
GRID = N_PAD // RB

_mesh = plsc.VectorSubcoreMesh(core_axis_name="c", subcore_axis_name="s")


def _zero16():
    return jnp.zeros((16,), jnp.float32)


@functools.partial(
    pl.kernel,
    out_type=jax.ShapeDtypeStruct((NC, N_PAD), jnp.float32),
    mesh=_mesh,
    compiler_params=pltpu.CompilerParams(use_tc_tiling_on_sc=False),
    scratch_types=[
        pltpu.VMEM((NCH, CHUNK), jnp.int32),
        pltpu.VMEM((CHUNK,), jnp.float32),
        pltpu.VMEM((ROWS_PER_TILE,), jnp.float32),
        pltpu.VMEM_SHARED((N_PAD,), jnp.float32),
    ],
)
def _deg_kernel(e_hbm, out_hbm, dst_v, ones_v, stage_v, acc_sh):
    c = lax.axis_index("c")
    s = lax.axis_index("s")
    wid = s * NC + c

    def fill(i, _):
        ones_v[pl.ds(i * 16, 16)] = _zero16() + 1.0
        return 0
    lax.fori_loop(0, CHUNK // 16, fill, 0)

    def zst(i, _):
        stage_v[pl.ds(i * 16, 16)] = _zero16()
        return 0
    lax.fori_loop(0, ROWS_PER_TILE // 16, zst, 0)
    pltpu.sync_copy(stage_v, acc_sh.at[pl.ds(s * ROWS_PER_TILE, ROWS_PER_TILE)])
    plsc.subcore_barrier()

    pltpu.sync_copy(e_hbm.at[1, pl.ds(wid * NCH, NCH)], dst_v)

    def body(ch, _):
        pltpu.sync_copy(ones_v, acc_sh.at[dst_v.at[ch]], add=True)
        return 0
    lax.fori_loop(0, NCH, body, 0)
    plsc.subcore_barrier()

    pltpu.sync_copy(acc_sh.at[pl.ds(s * ROWS_PER_TILE, ROWS_PER_TILE)], stage_v)
    pltpu.sync_copy(stage_v, out_hbm.at[c, pl.ds(s * ROWS_PER_TILE, ROWS_PER_TILE)])


def _make_agg(F):
    ncp = 1

    @functools.partial(
        pl.kernel,
        out_type=jax.ShapeDtypeStruct((NC, N_PAD, F), jnp.float32),
        mesh=_mesh,
        compiler_params=pltpu.CompilerParams(use_tc_tiling_on_sc=False),
        scratch_types=[
            pltpu.VMEM((NCH, CHUNK), jnp.int32),
            pltpu.VMEM((NCH, CHUNK), jnp.int32),
            [pltpu.VMEM((CHUNK, F), jnp.float32) for _ in range(BUFS)],
            pltpu.VMEM((ROWS_PER_TILE, F), jnp.float32),
            pltpu.VMEM((ROWS_PER_TILE, F), jnp.float32),
            pltpu.VMEM_SHARED((ncp, N_PAD, F), jnp.float32),
            [pltpu.SemaphoreType.DMA for _ in range(BUFS)],
            pltpu.SemaphoreType.DMA,
        ],
    )
    def agg(e_hbm, xp_hbm, out_hbm,
            src_v, dst_v, rows, stage_v, tmp_v, acc_sh, gsems, ssem):
        c = lax.axis_index("c")
        s = lax.axis_index("s")
        wid = s * NC + c
        grp = lax.rem(s, ncp)

        def zr(i, _):
            r = i // (F // 16)
            j = i % (F // 16)
            rows[0][r, pl.ds(j * 16, 16)] = _zero16()
            return 0
        lax.fori_loop(0, CHUNK * F // 16, zr, 0)
        for kc in range(ncp):
            for k in range(ROWS_PER_TILE // CHUNK):
                pltpu.sync_copy(
                    rows[0],
                    acc_sh.at[kc, pl.ds(s * ROWS_PER_TILE + k * CHUNK, CHUNK)])
        plsc.subcore_barrier()

        pltpu.sync_copy(e_hbm.at[0, pl.ds(wid * NCH, NCH)], src_v)
        pltpu.sync_copy(e_hbm.at[1, pl.ds(wid * NCH, NCH)], dst_v)

        def gather(ch, b):
            return pltpu.async_copy(xp_hbm.at[src_v.at[ch]], rows[b], gsems[b])

        def scat(ch, b):
            dst = acc_sh.at[grp].at[dst_v.at[ch]]
            pltpu.async_copy(rows[b], dst, ssem, add=True)
            pltpu.make_async_copy(rows[b], dst, ssem).wait()

        descs = {}
        for b in range(BUFS):
            descs[b] = gather(b, b)
        for ch in range(NCH):
            b = ch % BUFS
            descs[b].wait()
            scat(ch, b)
            if ch + BUFS < NCH:
                descs[b] = gather(ch + BUFS, b)
        plsc.subcore_barrier()

        rowslice = pl.ds(s * ROWS_PER_TILE, ROWS_PER_TILE)
        pltpu.sync_copy(acc_sh.at[0, rowslice], stage_v)
        for kc in range(1, ncp):
            pltpu.sync_copy(acc_sh.at[kc, rowslice], tmp_v)

            def addk(i, _):
                r = i // (F // 16)
                j = i % (F // 16)
                sl = pl.ds(j * 16, 16)
                stage_v[r, sl] = stage_v[r, sl] + tmp_v[r, sl]
                return 0
            lax.fori_loop(0, ROWS_PER_TILE * F // 16, addk, 0)
        pltpu.sync_copy(stage_v, out_hbm.at[c, rowslice])

    return agg


_agg32 = _make_agg(32)
_agg16 = _make_agg(16)


def _iota2(shape, dim):
    return lax.broadcasted_iota(jnp.int32, shape, dim)


def _dinv_col(deg_ref):
    deg = deg_ref[0:1, :] + deg_ref[1:2, :] + 1.0
    return jnp.transpose(lax.rsqrt(deg), (1, 0))


def _pack_rows(y, p, f):
    rr = RB // p
    r = _iota2((rr, RB), 0)
    n = _iota2((rr, RB), 1)
    outs = []
    for g in range(p):
        sel = jnp.where(n == p * r + g, 1.0, 0.0)
        outs.append(jnp.dot(sel, y, preferred_element_type=jnp.float32))
    return jnp.concatenate(outs, axis=1)


def _unpack_rows(pk, p, f):
    rr = RB // p
    n = _iota2((RB, rr), 0)
    r = _iota2((RB, rr), 1)
    acc = None
    for g in range(p):
        sel = jnp.where((r == n // p) & (n % p == g), 1.0, 0.0)
        t = jnp.dot(sel, pk[:, g * f:(g + 1) * f],
                    preferred_element_type=jnp.float32)
        acc = t if acc is None else acc + t
    return acc


def _prep_body(x_ref, w_ref, deg_ref, out_ref):
    dcol = _dinv_col(deg_ref)
    y = jnp.dot(x_ref[...], w_ref[...], preferred_element_type=jnp.float32)
    out_ref[...] = _pack_rows(y * dcol, 4, 32)


def _make_mid_body(p_in, f_in, p_out, f_out):
    def body(p_ref, xp_ref, deg_ref, b_ref, w_ref, out_ref):
        dcol = _dinv_col(deg_ref)
        tot = _unpack_rows(p_ref[0] + p_ref[1] + xp_ref[...], p_in, f_in)
        h = jnp.maximum(dcol * tot + b_ref[...], 0.0)
        y = jnp.dot(h, w_ref[...], preferred_element_type=jnp.float32) * dcol
        out_ref[...] = _pack_rows(y, p_out, f_out)
    return body


def _final_body(p_ref, xp_ref, deg_ref, b_ref, out_ref):
    dcol = _dinv_col(deg_ref)
    tot = _unpack_rows(p_ref[0] + p_ref[1] + xp_ref[...], 8, 16)
    z = dcol * tot + b_ref[...]
    col = _iota2(z.shape, 1)
    valid = col < 11
    zm = jnp.where(valid, z, -jnp.inf)
    m = jnp.max(zm, axis=1, keepdims=True)
    e = jnp.where(valid, jnp.exp(z - m), 0.0)
    out_ref[...] = z - m - jnp.log(jnp.sum(e, axis=1, keepdims=True))


def _deg_spec():
    return pl.BlockSpec((2, RB), lambda i: (0, i))


def _pk_spec(p):
    return pl.BlockSpec((RB // p * 128 // 128, 128), lambda i: (i, 0))


def _parts_pk_spec(p):
    return pl.BlockSpec((NC, RB // p, 128), lambda i: (0, i, 0))


def _full_spec(shape):
    return pl.BlockSpec(shape, lambda i: tuple(0 for _ in shape))


def _tc_prep(x_pad, W1, deg):
    return pl.pallas_call(
        _prep_body,
        grid=(GRID,),
        in_specs=[pl.BlockSpec((RB, 128), lambda i: (i, 0)),
                  _full_spec((128, 32)), _deg_spec()],
        out_specs=_pk_spec(4),
        out_shape=jax.ShapeDtypeStruct((N_PAD // 4, 128), jnp.float32),
    )(x_pad, W1, deg)


def _tc_mid(parts_pk, xp_pk, deg, b, Wn, p_in, f_in, p_out, f_out):
    return pl.pallas_call(
        _make_mid_body(p_in, f_in, p_out, f_out),
        grid=(GRID,),
        in_specs=[_parts_pk_spec(p_in), _pk_spec(p_in), _deg_spec(),
                  _full_spec((1, f_in)), _full_spec(Wn.shape)],
        out_specs=_pk_spec(p_out),
        out_shape=jax.ShapeDtypeStruct((N_PAD // p_out, 128), jnp.float32),
    )(parts_pk, xp_pk, deg, b.reshape(1, f_in), Wn)


def _tc_final(parts_pk, xp_pk, deg, b):
    return pl.pallas_call(
        _final_body,
        grid=(GRID,),
        in_specs=[_parts_pk_spec(8), _pk_spec(8), _deg_spec(),
                  _full_spec((1, 16))],
        out_specs=pl.BlockSpec((RB, 16), lambda i: (i, 0)),
        out_shape=jax.ShapeDtypeStruct((N_PAD, 16), jnp.float32),
    )(parts_pk, xp_pk, deg, b.reshape(1, 16))


@jax.jit
def kernel(x, edge_index, W1, b1, W2, b2, W3, b3):
    ei = edge_index.astype(jnp.int32)
    e3 = jnp.pad(ei, ((0, 0), (0, E_PAD - N_EDGES)),
                 constant_values=N_NODES).reshape(2, NW * NCH, CHUNK)

    x_pad = jnp.pad(x, ((0, N_PAD - N_NODES), (0, 0)))
    W3p = jnp.pad(W3, ((0, 0), (0, 5)))
    b3p = jnp.pad(b3, (0, 5))

    deg = _deg_kernel(e3)

    x1p = _tc_prep(x_pad, W1, deg)
    parts1 = _agg32(e3, x1p.reshape(N_PAD, 32))
    x2p = _tc_mid(parts1.reshape(2, N_PAD // 4, 128), x1p, deg,
                  b1, W2, 4, 32, 8, 16)
    parts2 = _agg16(e3, x2p.reshape(N_PAD, 16))
    x3p = _tc_mid(parts2.reshape(2, N_PAD // 8, 128), x2p, deg,
                  b2, W3p, 8, 16, 8, 16)
    parts3 = _agg16(e3, x3p.reshape(N_PAD, 16))
    out = _tc_final(parts3.reshape(2, N_PAD // 8, 128), x3p, deg, b3p)
    return out[:N_NODES, :11]
